# Initial kernel scaffold; baseline (speedup 1.0000x reference)
import jax
import jax.numpy as jnp
from jax.experimental import pallas as pl


def kernel(node, X, edge_index, W_iv, b_iv, W_in, b_in, W_h, b_h, W_out, b_out, W_il, b_il, W_io, b_io):
    raise NotImplementedError("write your pallas kernel here")


# speedup vs baseline: 7.8592x; 7.8592x over previous
import functools

import jax
import jax.numpy as jnp
import numpy as np
from jax import lax
from jax.experimental import pallas as pl
from jax.experimental.pallas import tpu as pltpu
from jax.experimental.pallas import tpu_sc as plsc

NC = 2
NS = 16
NW = NC * NS
N_NODES = 10000
D = 128
CHUNK = 64
G = 32
NPAD_ROWS = 112
NROWS = N_NODES + NPAD_ROWS
ROWS_PER_TILE = NROWS // NS


@functools.partial(jax.jit, static_argnames=("ngroups",))
def _segment_sum_sc(h, srcd, zeros_tile, *, ngroups):
    @functools.partial(
        pl.kernel,
        out_type=jax.ShapeDtypeStruct((NC, NROWS, D), jnp.float32),
        mesh=plsc.VectorSubcoreMesh(
            core_axis_name="c", subcore_axis_name="s",
            num_cores=NC, num_subcores=NS),
        scratch_types=[
            pltpu.VMEM((2, G, CHUNK), jnp.int32),
            pltpu.VMEM((CHUNK, D), jnp.float32),
            pltpu.VMEM((CHUNK, D), jnp.float32),
            pltpu.SemaphoreType.DMA,
            pltpu.SemaphoreType.DMA,
            pltpu.VMEM_SHARED((NROWS, D), jnp.float32),
        ],
    )
    def seg_kernel(h_hbm, srcd_hbm, zeros_hbm, out_hbm,
                   idx_v, buf_a, buf_b, sem_a, sem_b, agg_sh):
        c = lax.axis_index("c")
        s = lax.axis_index("s")
        wid = c * NS + s
        pltpu.sync_copy(zeros_hbm, agg_sh.at[pl.ds(s * ROWS_PER_TILE, ROWS_PER_TILE)])
        plsc.subcore_barrier()

        def group_body(g, _):
            pltpu.sync_copy(srcd_hbm.at[wid, g], idx_v)
            pltpu.async_copy(h_hbm.at[idx_v.at[0, 0]], buf_a, sem_a)

            def body(jj, _):
                @pl.when(lax.rem(jj, 2) == 0)
                def _even():
                    pltpu.make_async_copy(h_hbm.at[idx_v.at[0, jj]], buf_a, sem_a).wait()
                    @pl.when(jj + 1 < G)
                    def _():
                        pltpu.async_copy(h_hbm.at[idx_v.at[0, jj + 1]], buf_b, sem_b)
                    pltpu.sync_copy(buf_a, agg_sh.at[idx_v.at[1, jj]], add=True)

                @pl.when(lax.rem(jj, 2) == 1)
                def _odd():
                    pltpu.make_async_copy(h_hbm.at[idx_v.at[0, jj]], buf_b, sem_b).wait()
                    @pl.when(jj + 1 < G)
                    def _():
                        pltpu.async_copy(h_hbm.at[idx_v.at[0, jj + 1]], buf_a, sem_a)
                    pltpu.sync_copy(buf_b, agg_sh.at[idx_v.at[1, jj]], add=True)
                return 0

            lax.fori_loop(0, G, body, 0)
            return 0

        lax.fori_loop(0, ngroups, group_body, 0)
        plsc.subcore_barrier()
        pltpu.sync_copy(
            agg_sh.at[pl.ds(s * ROWS_PER_TILE, ROWS_PER_TILE)],
            out_hbm.at[c, pl.ds(s * ROWS_PER_TILE, ROWS_PER_TILE)])

    return seg_kernel(h, srcd, zeros_tile)


def _u_body(node_ref, wiv_ref, biv_ref, wtop_ref, out_ref):
    x = jnp.dot(node_ref[...], wiv_ref[...],
                preferred_element_type=jnp.float32) + biv_ref[...]
    out_ref[...] = jnp.dot(x, wtop_ref[...], preferred_element_type=jnp.float32)


def _compute_u(node, W_iv, b_iv, W_il_top):
    B = node.shape[0]
    K = node.shape[1]
    BB = 256
    return pl.pallas_call(
        _u_body,
        grid=(B // BB,),
        in_specs=[
            pl.BlockSpec((BB, K), lambda i: (i, 0)),
            pl.BlockSpec((K, D), lambda i: (0, 0)),
            pl.BlockSpec((1, D), lambda i: (0, 0)),
            pl.BlockSpec((D, D), lambda i: (0, 0)),
        ],
        out_specs=pl.BlockSpec((BB, D), lambda i: (i, 0)),
        out_shape=jax.ShapeDtypeStruct((B, D), jnp.float32),
    )(node, W_iv, b_iv.reshape(1, D), W_il_top)


def _layer_body(p0_ref, p1_ref, w_ref, b_ref, out_ref, *, act):
    a = p0_ref[0] + p1_ref[0]
    t = jnp.dot(a, w_ref[...], preferred_element_type=jnp.float32) + b_ref[...]
    if act == "elu":
        t = jnp.where(t > 0, t, jnp.exp(jnp.minimum(t, 0.0)) - 1.0)
    else:
        t = jnp.maximum(t, 0.0)
    out_ref[...] = t


def _layer(agg_pair, W, b, act):
    R = 2528
    grid = NROWS // R
    return pl.pallas_call(
        functools.partial(_layer_body, act=act),
        grid=(grid,),
        in_specs=[
            pl.BlockSpec((1, R, D), lambda i: (0, i, 0)),
            pl.BlockSpec((1, R, D), lambda i: (1, i, 0)),
            pl.BlockSpec((D, D), lambda i: (0, 0)),
            pl.BlockSpec((1, D), lambda i: (0, 0)),
        ],
        out_specs=pl.BlockSpec((R, D), lambda i: (i, 0)),
        out_shape=jax.ShapeDtypeStruct((NROWS, D), jnp.float32),
    )(agg_pair, agg_pair, W, b.reshape(1, D))


def _layer3_body(p0_ref, p1_ref, w_ref, b_ref, out_ref, *, rows):
    i = pl.program_id(0)
    a = p0_ref[0] + p1_ref[0]
    t = jnp.dot(a, w_ref[...], preferred_element_type=jnp.float32) + b_ref[...]
    t = jnp.maximum(t, 0.0)
    rid = i * rows + lax.broadcasted_iota(jnp.int32, (rows, 1), 0)
    t = jnp.where(rid < N_NODES, t, 0.0)
    part = jnp.sum(t, axis=0, keepdims=True)

    @pl.when(i == 0)
    def _():
        out_ref[...] = jnp.zeros_like(out_ref)

    out_ref[...] += part


def _layer3_colsum(agg_pair, W, b):
    R = 2528
    grid = NROWS // R
    return pl.pallas_call(
        functools.partial(_layer3_body, rows=R),
        grid=(grid,),
        in_specs=[
            pl.BlockSpec((1, R, D), lambda i: (0, i, 0)),
            pl.BlockSpec((1, R, D), lambda i: (1, i, 0)),
            pl.BlockSpec((D, D), lambda i: (0, 0)),
            pl.BlockSpec((1, D), lambda i: (0, 0)),
        ],
        out_specs=pl.BlockSpec((1, D), lambda i: (0, 0)),
        out_shape=jax.ShapeDtypeStruct((1, D), jnp.float32),
    )(agg_pair, agg_pair, W, b.reshape(1, D))


def _final_body(u_ref, ysum_ref, wbot_ref, bil_ref, wio_ref, bio_ref, out_ref):
    ymean = ysum_ref[...] * np.float32(1.0 / N_NODES)
    cvec = jnp.dot(ymean, wbot_ref[...], preferred_element_type=jnp.float32) \
        + bil_ref[...]
    z = jnp.maximum(u_ref[...] + cvec, 0.0)
    out_ref[...] = jnp.dot(z, wio_ref[...], preferred_element_type=jnp.float32) \
        + bio_ref[...]


def _final(u, ysum, W_il_bot, b_il, W_io_pad, b_io_pad):
    B = u.shape[0]
    return pl.pallas_call(
        _final_body,
        out_shape=jax.ShapeDtypeStruct((B, D), jnp.float32),
    )(u, ysum, W_il_bot, b_il.reshape(1, D), W_io_pad, b_io_pad)


def kernel(node, X, edge_index, W_iv, b_iv, W_in, b_in, W_h, b_h,
           W_out, b_out, W_il, b_il, W_io, b_io):
    E = edge_index.shape[1]
    src = edge_index[0].astype(jnp.int32)
    dst = edge_index[1].astype(jnp.int32)
    ngroups = -(-E // (NW * G * CHUNK))
    ep = NW * ngroups * G * CHUNK
    pad = ep - E
    if pad:
        padi = jnp.arange(pad, dtype=jnp.int32)
        src = jnp.concatenate([src, padi % N_NODES])
        dst = jnp.concatenate([dst, N_NODES + padi % NPAD_ROWS])
    srcd = jnp.stack([src.reshape(NW, ngroups, G, CHUNK),
                      dst.reshape(NW, ngroups, G, CHUNK)], axis=2)
    zeros_tile = jnp.zeros((ROWS_PER_TILE, D), jnp.float32)

    u = _compute_u(node, W_iv, b_iv, W_il[:D])

    agg1 = _segment_sum_sc(X, srcd, zeros_tile, ngroups=ngroups)
    h1 = _layer(agg1, W_in, b_in, "elu")
    agg2 = _segment_sum_sc(h1, srcd, zeros_tile, ngroups=ngroups)
    h2 = _layer(agg2, W_h, b_h, "relu")
    agg3 = _segment_sum_sc(h2, srcd, zeros_tile, ngroups=ngroups)
    ysum = _layer3_colsum(agg3, W_out, b_out)

    W_io_pad = jnp.pad(W_io, ((0, 0), (0, D - W_io.shape[1])))
    b_io_pad = jnp.pad(b_io, (0, D - b_io.shape[0])).reshape(1, D)
    out = _final(u, ysum, W_il[D:], b_il, W_io_pad, b_io_pad)
    return out[:, :1]
